# Initial kernel scaffold; baseline (speedup 1.0000x reference)
import jax
import jax.numpy as jnp
from jax.experimental import pallas as pl


def kernel(x, W, b):
    raise NotImplementedError("write your pallas kernel here")


# speedup vs baseline: 2.5605x; 2.5605x over previous
import functools

import jax
import jax.numpy as jnp
from jax.experimental import pallas as pl

BLOCK_ROWS = 512
N_EXPERTS = 16
TOPK = 2


def _gate_block(x_ref, w_ref, b_ref, gates_ref, idx_ref):
    logits = jnp.dot(x_ref[...], w_ref[...],
                     preferred_element_type=jnp.float32) + b_ref[...]
    rows = logits.shape[0]
    lane = jax.lax.broadcasted_iota(jnp.int32, (rows, N_EXPERTS), 1)

    m1 = jnp.max(logits, axis=1, keepdims=True)
    i1 = jnp.min(jnp.where(logits == m1, lane, N_EXPERTS), axis=1,
                 keepdims=True)
    masked = jnp.where(lane == i1, -jnp.inf, logits)
    m2 = jnp.max(masked, axis=1, keepdims=True)
    i2 = jnp.min(jnp.where(masked == m2, lane, N_EXPERTS), axis=1,
                 keepdims=True)

    g1 = jax.nn.sigmoid(m1 - m2)
    g2 = 1.0 - g1
    gates_ref[...] = (jnp.where(lane == i1, g1, 0.0)
                      + jnp.where(lane == i2, g2, 0.0))

    pair = jax.lax.broadcasted_iota(jnp.int32, (rows, TOPK), 1)
    idx_ref[...] = jnp.where(pair == 0, i1, i2)


@functools.partial(jax.jit, static_argnames=("interpret",))
def kernel(x, W, b, interpret=False):
    x = x.astype(jnp.float32)
    rows = x.shape[0]
    b2 = b.reshape(1, N_EXPERTS).astype(jnp.float32)
    grid = (rows // BLOCK_ROWS,)
    gates, idx = pl.pallas_call(
        _gate_block,
        grid=grid,
        in_specs=[
            pl.BlockSpec((BLOCK_ROWS, x.shape[1]), lambda i: (i, 0)),
            pl.BlockSpec((x.shape[1], N_EXPERTS), lambda i: (0, 0)),
            pl.BlockSpec((1, N_EXPERTS), lambda i: (0, 0)),
        ],
        out_specs=[
            pl.BlockSpec((BLOCK_ROWS, N_EXPERTS), lambda i: (i, 0)),
            pl.BlockSpec((BLOCK_ROWS, TOPK), lambda i: (i, 0)),
        ],
        out_shape=[
            jax.ShapeDtypeStruct((rows, N_EXPERTS), jnp.float32),
            jax.ShapeDtypeStruct((rows, TOPK), jnp.int32),
        ],
        interpret=interpret,
    )(x, W.astype(jnp.float32), b2)
    return (gates, idx)


# speedup vs baseline: 3.1026x; 1.2117x over previous
import functools

import jax
import jax.numpy as jnp
from jax.experimental import pallas as pl

BLOCK_ROWS = 2048
N_EXPERTS = 16
TOPK = 2


def _gate_block(x_ref, w_ref, b_ref, gates_ref, idx_ref):
    logits = jnp.dot(x_ref[...], w_ref[...],
                     preferred_element_type=jnp.float32) + b_ref[...]
    rows = logits.shape[0]
    lane = jax.lax.broadcasted_iota(jnp.int32, (rows, N_EXPERTS), 1)

    m1 = jnp.max(logits, axis=1, keepdims=True)
    i1 = jnp.min(jnp.where(logits == m1, lane, N_EXPERTS), axis=1,
                 keepdims=True)
    masked = jnp.where(lane == i1, -jnp.inf, logits)
    m2 = jnp.max(masked, axis=1, keepdims=True)
    i2 = jnp.min(jnp.where(masked == m2, lane, N_EXPERTS), axis=1,
                 keepdims=True)

    g1 = jax.nn.sigmoid(m1 - m2)
    g2 = 1.0 - g1
    gates_ref[...] = (jnp.where(lane == i1, g1, 0.0)
                      + jnp.where(lane == i2, g2, 0.0))

    pair = jax.lax.broadcasted_iota(jnp.int32, (rows, TOPK), 1)
    idx_ref[...] = jnp.where(pair == 0, i1, i2)


@functools.partial(jax.jit, static_argnames=("interpret",))
def kernel(x, W, b, interpret=False):
    x = x.astype(jnp.float32)
    rows = x.shape[0]
    b2 = b.reshape(1, N_EXPERTS).astype(jnp.float32)
    grid = (rows // BLOCK_ROWS,)
    gates, idx = pl.pallas_call(
        _gate_block,
        grid=grid,
        in_specs=[
            pl.BlockSpec((BLOCK_ROWS, x.shape[1]), lambda i: (i, 0)),
            pl.BlockSpec((x.shape[1], N_EXPERTS), lambda i: (0, 0)),
            pl.BlockSpec((1, N_EXPERTS), lambda i: (0, 0)),
        ],
        out_specs=[
            pl.BlockSpec((BLOCK_ROWS, N_EXPERTS), lambda i: (i, 0)),
            pl.BlockSpec((BLOCK_ROWS, TOPK), lambda i: (i, 0)),
        ],
        out_shape=[
            jax.ShapeDtypeStruct((rows, N_EXPERTS), jnp.float32),
            jax.ShapeDtypeStruct((rows, TOPK), jnp.int32),
        ],
        interpret=interpret,
    )(x, W.astype(jnp.float32), b2)
    return (gates, idx)
